# baseline (speedup 1.0000x reference)
import functools

import jax
import jax.numpy as jnp
from jax.experimental import pallas as pl

B, C, N = 2, 6, 4096
K_TOP = 64
NSAMPLE = 32
NCAND = 552


def _fe_body(x_ref, W1_ref, b1_ref, W2_ref, b2_ref, W3_ref, b3_ref,
             Ww1_ref, bw1_ref, Ww2_ref, bw2_ref, feats_ref, scores_ref):
    x = x_ref[0]
    h = jax.nn.relu(jnp.dot(x, W1_ref[...], preferred_element_type=jnp.float32) + b1_ref[...])
    h = jax.nn.relu(jnp.dot(h, W2_ref[...], preferred_element_type=jnp.float32) + b2_ref[...])
    feats = jnp.dot(h, W3_ref[...], preferred_element_type=jnp.float32) + b3_ref[...]
    s = jax.nn.relu(jnp.dot(feats, Ww1_ref[...], preferred_element_type=jnp.float32) + bw1_ref[...])
    s = jnp.dot(s, Ww2_ref[...], preferred_element_type=jnp.float32) + bw2_ref[...]
    feats_ref[0] = feats
    scores_ref[0] = s[:, 0:1]


def _fe_pallas(x_all, W1, b1, W2, b2, W3, b3, Ww1, bw1, Ww2, bw2):
    G = x_all.shape[0]
    grid = (G,)
    out = pl.pallas_call(
        _fe_body,
        grid=grid,
        in_specs=[
            pl.BlockSpec((1, N, C), lambda g: (g, 0, 0)),
            pl.BlockSpec((C, 64), lambda g: (0, 0)),
            pl.BlockSpec((1, 64), lambda g: (0, 0)),
            pl.BlockSpec((64, 64), lambda g: (0, 0)),
            pl.BlockSpec((1, 64), lambda g: (0, 0)),
            pl.BlockSpec((64, 32), lambda g: (0, 0)),
            pl.BlockSpec((1, 32), lambda g: (0, 0)),
            pl.BlockSpec((32, 16), lambda g: (0, 0)),
            pl.BlockSpec((1, 16), lambda g: (0, 0)),
            pl.BlockSpec((16, 1), lambda g: (0, 0)),
            pl.BlockSpec((1, 1), lambda g: (0, 0)),
        ],
        out_specs=[
            pl.BlockSpec((1, N, 32), lambda g: (g, 0, 0)),
            pl.BlockSpec((1, N, 1), lambda g: (g, 0, 0)),
        ],
        out_shape=[
            jax.ShapeDtypeStruct((G, N, 32), jnp.float32),
            jax.ShapeDtypeStruct((G, N, 1), jnp.float32),
        ],
    )(x_all, W1, b1.reshape(1, 64), W2, b2.reshape(1, 64), W3, b3.reshape(1, 32),
      Ww1, bw1.reshape(1, 16), Ww2, bw2.reshape(1, 1))
    return out


def kernel(src_pts, tgt_pts, candidate_pts, W1, b1, W2, b2, W3, b3,
           Ww1, bw1, Ww2, bw2, Wsrc, bsrc, Wtgt, btgt):
    src_t = jnp.transpose(src_pts, (0, 2, 1))
    tgt_t = jnp.transpose(tgt_pts, (0, 2, 1))
    x_all = jnp.concatenate([src_t, tgt_t], axis=0)
    feats_all, scores_all = _fe_pallas(x_all, W1, b1, W2, b2, W3, b3, Ww1, bw1, Ww2, bw2)
    src_feats = feats_all[:B]
    tgt_feats = feats_all[B:]
    scores = scores_all[:B, :, 0]

    _, keyidx = jax.lax.top_k(scores, K_TOP)
    src_keypts = jnp.take_along_axis(src_t, keyidx[:, :, None], axis=1)
    key_xyz = src_keypts[:, :, :3]

    d2 = jnp.sum((key_xyz[:, :, None, :] - key_xyz[:, None, :, :]) ** 2, axis=-1)
    _, picked_idx = jax.lax.top_k(-d2, NSAMPLE)
    grouped_xyz = jax.vmap(lambda xb, ib: xb[ib])(key_xyz, picked_idx)
    rel_xyz = grouped_xyz - key_xyz[:, :, None, :]
    key_feats = jnp.take_along_axis(src_feats, keyidx[:, :, None], axis=1)
    grouped_feats = jax.vmap(lambda fb, ib: fb[ib])(key_feats, picked_idx)
    cat = jnp.concatenate([rel_xyz, grouped_feats], axis=-1)
    src_keyfeats_cat = jnp.max(jax.nn.relu(cat @ Wsrc + bsrc), axis=2)

    tgt_xyz = tgt_t[:, :, :3]

    def cand_one(cand, xyz_b, feats_b):
        d = jnp.sum((cand[:, None, :] - xyz_b[None, :, :]) ** 2, axis=-1)
        _, idx = jax.lax.top_k(-d, NSAMPLE)
        gx = xyz_b[idx]
        rel = gx - cand[:, None, :]
        gf = feats_b[idx]
        c = jnp.concatenate([rel, gf], axis=-1)
        return jnp.max(jax.nn.relu(c @ Wtgt + btgt), axis=1)

    outs = []
    for b in range(B):
        ob = jax.lax.map(lambda cnd, xb=tgt_xyz[b], fb=tgt_feats[b]: cand_one(cnd, xb, fb), candidate_pts[b])
        outs.append(ob)
    tgt_keyfeats_cat = jnp.stack(outs, axis=0)

    src_keypts = src_keypts + 0.0 * jnp.sum(src_keyfeats_cat) + 0.0 * jnp.sum(tgt_keyfeats_cat)
    return src_keypts


# speedup vs baseline: 3.9267x; 3.9267x over previous
import functools

import jax
import jax.numpy as jnp
from jax import lax
from jax.experimental import pallas as pl
from jax.experimental.pallas import tpu as pltpu, tpu_sc as plsc

B, C, N = 2, 6, 4096
K_TOP = 64
NSAMPLE = 32
NCAND = 552
Q = K_TOP * NCAND
NWORK = 16
QPW = Q // NWORK
NCHUNK = QPW // 16
NPC = N // 16


def _fe_body(x_ref, W1_ref, b1_ref, W2_ref, b2_ref, W3_ref, b3_ref,
             Ww1_ref, bw1_ref, Ww2_ref, bw2_ref, Wp_ref,
             feats_ref, scores_ref, proj_ref):
    x = x_ref[0]
    h = jax.nn.relu(jnp.dot(x, W1_ref[...], preferred_element_type=jnp.float32) + b1_ref[...])
    h = jax.nn.relu(jnp.dot(h, W2_ref[...], preferred_element_type=jnp.float32) + b2_ref[...])
    feats = jnp.dot(h, W3_ref[...], preferred_element_type=jnp.float32) + b3_ref[...]
    s = jax.nn.relu(jnp.dot(feats, Ww1_ref[...], preferred_element_type=jnp.float32) + bw1_ref[...])
    s = jnp.dot(s, Ww2_ref[...], preferred_element_type=jnp.float32) + bw2_ref[...]
    cat = jnp.concatenate([x[:, :3], feats], axis=1)
    catp = jnp.pad(cat, ((0, 0), (0, 5)))
    proj = jnp.dot(catp, Wp_ref[0], preferred_element_type=jnp.float32)
    feats_ref[0] = feats
    scores_ref[0] = s[:, 0:1]
    proj_ref[0] = proj


def _fe_pallas(x_all, W1, b1, W2, b2, W3, b3, Ww1, bw1, Ww2, bw2, Wp):
    G = x_all.shape[0]
    return pl.pallas_call(
        _fe_body,
        grid=(G,),
        in_specs=[
            pl.BlockSpec((1, N, C), lambda g: (g, 0, 0)),
            pl.BlockSpec((C, 64), lambda g: (0, 0)),
            pl.BlockSpec((1, 64), lambda g: (0, 0)),
            pl.BlockSpec((64, 64), lambda g: (0, 0)),
            pl.BlockSpec((1, 64), lambda g: (0, 0)),
            pl.BlockSpec((64, 32), lambda g: (0, 0)),
            pl.BlockSpec((1, 32), lambda g: (0, 0)),
            pl.BlockSpec((32, 16), lambda g: (0, 0)),
            pl.BlockSpec((1, 16), lambda g: (0, 0)),
            pl.BlockSpec((16, 1), lambda g: (0, 0)),
            pl.BlockSpec((1, 1), lambda g: (0, 0)),
            pl.BlockSpec((1, 40, 32), lambda g: (g // B, 0, 0)),
        ],
        out_specs=[
            pl.BlockSpec((1, N, 32), lambda g: (g, 0, 0)),
            pl.BlockSpec((1, N, 1), lambda g: (g, 0, 0)),
            pl.BlockSpec((1, N, 32), lambda g: (g, 0, 0)),
        ],
        out_shape=[
            jax.ShapeDtypeStruct((G, N, 32), jnp.float32),
            jax.ShapeDtypeStruct((G, N, 1), jnp.float32),
            jax.ShapeDtypeStruct((G, N, 32), jnp.float32),
        ],
    )(x_all, W1, b1.reshape(1, 64), W2, b2.reshape(1, 64), W3, b3.reshape(1, 32),
      Ww1, bw1.reshape(1, 16), Ww2, bw2.reshape(1, 1), Wp)


_CP = pltpu.CompilerParams(needs_layout_passes=False, use_tc_tiling_on_sc=False)


def _find_bin(bins_ref, nvreg, target, scr16):
    carry = jnp.zeros((16,), jnp.int32)
    found = jnp.zeros((16,), jnp.int32)
    pbin = jnp.zeros((16,), jnp.int32)
    cbelow = jnp.zeros((16,), jnp.int32)
    iota = lax.iota(jnp.int32, 16)
    for j in range(nvreg):
        v = bins_ref[pl.ds(j * 16, 16)]
        incl = plsc.cumsum(v) + carry
        excl = incl - v
        m = (incl >= target) & (found == 0)
        ff = plsc.all_reduce_ffs(m)
        has = (ff < 16) & (found == 0)
        scr16[...] = excl
        e = plsc.load_gather(scr16, [jnp.minimum(ff, 15)])
        pbin = jnp.where(has, j * 16 + ff, pbin)
        cbelow = jnp.where(has, e, cbelow)
        found = jnp.where(has, 1, found)
        scr16[...] = incl
        carry = plsc.load_gather(scr16, [jnp.full((16,), 15, jnp.int32)])
    return pbin, cbelow


def _sc_knn_kernel(xyz_hbm, tpack_hbm, cand_hbm, wt_hbm, out_hbm,
                   xv, tpk, candv, wtv, dbits, bins8, bins2, nbr, outbuf, scr16):
    b = lax.axis_index("c")
    w = lax.axis_index("s")
    iota = lax.iota(jnp.int32, 16)
    ones = jnp.ones((16,), jnp.int32)
    zeros = jnp.zeros((16,), jnp.int32)

    pltpu.sync_copy(xyz_hbm.at[b], xv)
    pltpu.sync_copy(tpack_hbm.at[b], tpk)
    for i in range(3):
        pltpu.sync_copy(cand_hbm.at[b, i, pl.ds(w * QPW, QPW)], candv.at[i])
    pltpu.sync_copy(wt_hbm, wtv)

    def per_query(q_local, qq):
        qidx = jnp.full((16,), q_local, jnp.int32)
        c0 = plsc.load_gather(candv, [zeros, qidx])
        c1 = plsc.load_gather(candv, [ones, qidx])
        c2 = plsc.load_gather(candv, [ones + ones, qidx])
        cc = c0 * c0 + c1 * c1 + c2 * c2

        zf = jnp.zeros((16,), jnp.int32)
        for j in range(8):
            bins8[pl.ds(j * 16, 16)] = zf
        for j in range(16):
            bins2[pl.ds(j * 16, 16)] = zf

        def passA(j, carry):
            x0 = xv[0, pl.ds(j * 16, 16)]
            x1 = xv[1, pl.ds(j * 16, 16)]
            x2 = xv[2, pl.ds(j * 16, 16)]
            s = xv[3, pl.ds(j * 16, 16)]
            t = c0 * x0 + c1 * x1 + c2 * x2
            d = jnp.maximum((s + cc) - (t + t), 0.0)
            bits = plsc.bitcast(d, jnp.int32)
            dbits[pl.ds(j * 16, 16)] = bits
            plsc.addupdate_scatter(bins8, [jnp.right_shift(bits, 24)], ones)
            return carry
        lax.fori_loop(0, NPC, passA, 0)

        p8, clt8 = _find_bin(bins8, 8, jnp.full((16,), NSAMPLE, jnp.int32), scr16)

        def passB(j, carry):
            bits = dbits[pl.ds(j * 16, 16)]
            m = jnp.right_shift(bits, 24) == p8
            sub = jnp.bitwise_and(jnp.right_shift(bits, 16), 255)
            plsc.addupdate_scatter(bins2, [sub], ones, mask=m)
            return carry
        lax.fori_loop(0, NPC, passB, 0)
        t16 = jnp.full((16,), NSAMPLE, jnp.int32) - clt8
        p16, clt16 = _find_bin(bins2, 16, t16, scr16)
        pfx16 = jnp.bitwise_or(jnp.left_shift(p8, 8), p16)
        clt = clt8 + clt16

        for j in range(16):
            bins2[pl.ds(j * 16, 16)] = zf

        def passC(j, carry):
            bits = dbits[pl.ds(j * 16, 16)]
            m = jnp.right_shift(bits, 16) == pfx16
            sub = jnp.bitwise_and(jnp.right_shift(bits, 8), 255)
            plsc.addupdate_scatter(bins2, [sub], ones, mask=m)
            return carry
        lax.fori_loop(0, NPC, passC, 0)
        t24 = jnp.full((16,), NSAMPLE, jnp.int32) - clt
        p24, _ = _find_bin(bins2, 16, t24, scr16)
        pfx24 = jnp.bitwise_or(jnp.left_shift(pfx16, 8), p24)

        def passD(j, cnt):
            bits = dbits[pl.ds(j * 16, 16)]
            m = jnp.right_shift(bits, 8) <= pfx24
            mi = jnp.where(m, 1, 0)
            pos = cnt + plsc.cumsum(mi) - mi
            plsc.store_scatter(nbr, [pos], j * 16 + iota, mask=m & (pos < NSAMPLE))
            return cnt + plsc.all_reduce_population_count(m)
        lax.fori_loop(0, NPC, passD, jnp.zeros((16,), jnp.int32))

        acc_e = jnp.full((16,), -3.0e38, jnp.float32)
        acc_o = jnp.full((16,), -3.0e38, jnp.float32)
        for j in range(NSAMPLE):
            ridx = plsc.load_gather(nbr, [jnp.full((16,), j, jnp.int32)])
            word = plsc.load_gather(tpk, [ridx, iota])
            a, o = plsc.unpack(plsc.bitcast(word, jnp.bfloat16),
                               format=plsc.PackFormat.INTERLEAVED)
            acc_e = jnp.maximum(acc_e, a.astype(jnp.float32))
            acc_o = jnp.maximum(acc_o, o.astype(jnp.float32))

        w0e = wtv[0, pl.ds(0, 16)]
        w0o = wtv[0, pl.ds(16, 16)]
        w1e = wtv[1, pl.ds(0, 16)]
        w1o = wtv[1, pl.ds(16, 16)]
        w2e = wtv[2, pl.ds(0, 16)]
        w2o = wtv[2, pl.ds(16, 16)]
        bte = wtv[3, pl.ds(0, 16)]
        bto = wtv[3, pl.ds(16, 16)]
        qe = bte - (c0 * w0e + c1 * w1e + c2 * w2e)
        qo = bto - (c0 * w0o + c1 * w1o + c2 * w2o)
        oe = jnp.maximum(acc_e + qe, 0.0)
        oo = jnp.maximum(acc_o + qo, 0.0)
        outbuf[pl.ds(qq * 32, 16)] = oe
        outbuf[pl.ds(qq * 32 + 16, 16)] = oo

    def chunk_body(ch, carry):
        def inner(qq, carry2):
            per_query(ch * 16 + qq, qq)
            return carry2
        lax.fori_loop(0, 16, inner, 0)
        pltpu.sync_copy(outbuf, out_hbm.at[b, pl.ds((w * QPW + ch * 16) * 32, 512)])
        return carry
    lax.fori_loop(0, NCHUNK, chunk_body, 0)


@functools.lru_cache(maxsize=1)
def _get_sc_knn():
    mesh = plsc.VectorSubcoreMesh(core_axis_name="c", subcore_axis_name="s")

    @functools.partial(pl.kernel,
                       out_type=jax.ShapeDtypeStruct((B, Q * 32), jnp.float32),
                       mesh=mesh, compiler_params=_CP,
                       scratch_types=[
                           pltpu.VMEM((4, N), jnp.float32),
                           pltpu.VMEM((N, 16), jnp.int32),
                           pltpu.VMEM((3, QPW), jnp.float32),
                           pltpu.VMEM((4, 32), jnp.float32),
                           pltpu.VMEM((N,), jnp.int32),
                           pltpu.VMEM((128,), jnp.int32),
                           pltpu.VMEM((256,), jnp.int32),
                           pltpu.VMEM((NSAMPLE,), jnp.int32),
                           pltpu.VMEM((512,), jnp.float32),
                           pltpu.VMEM((16,), jnp.int32),
                       ])
    def _sc_knn(xyz_hbm, tpack_hbm, cand_hbm, wt_hbm, out_hbm, *scr):
        _sc_knn_kernel(xyz_hbm, tpack_hbm, cand_hbm, wt_hbm, out_hbm, *scr)

    return _sc_knn


def kernel(src_pts, tgt_pts, candidate_pts, W1, b1, W2, b2, W3, b3,
           Ww1, bw1, Ww2, bw2, Wsrc, bsrc, Wtgt, btgt):
    src_t = jnp.transpose(src_pts, (0, 2, 1))
    tgt_t = jnp.transpose(tgt_pts, (0, 2, 1))
    x_all = jnp.concatenate([src_t, tgt_t], axis=0)
    Wp = jnp.stack([jnp.pad(Wsrc, ((0, 5), (0, 0))),
                    jnp.pad(Wtgt, ((0, 5), (0, 0)))], axis=0)
    feats_all, scores_all, proj_all = _fe_pallas(
        x_all, W1, b1, W2, b2, W3, b3, Ww1, bw1, Ww2, bw2, Wp)
    src_feats = feats_all[:B]
    scores = scores_all[:B, :, 0]

    _, keyidx = jax.lax.top_k(scores, K_TOP)
    src_keypts = jnp.take_along_axis(src_t, keyidx[:, :, None], axis=1)
    key_xyz = src_keypts[:, :, :3]

    d2 = jnp.sum((key_xyz[:, :, None, :] - key_xyz[:, None, :, :]) ** 2, axis=-1)
    _, picked_idx = jax.lax.top_k(-d2, NSAMPLE)
    proj_src = jnp.take_along_axis(proj_all[:B], keyidx[:, :, None], axis=1)
    gathered = jax.vmap(lambda pb, ib: pb[ib])(proj_src, picked_idx)
    qoff_src = bsrc - jnp.einsum("bkc,cj->bkj", key_xyz, Wsrc[:3])
    src_keyfeats_cat = jax.nn.relu(jnp.max(gathered, axis=2) + qoff_src)

    tgt_xyz_sq = jnp.sum(tgt_t[:, :, :3] ** 2, axis=-1)
    xyzcat = jnp.concatenate([tgt_pts[:, :3], tgt_xyz_sq[:, None]], 1)
    proj_tgt = proj_all[B:]
    tpack = jax.lax.bitcast_convert_type(
        proj_tgt.astype(jnp.bfloat16).reshape(B, N, 16, 2), jnp.int32)
    cand_flat = jnp.transpose(candidate_pts.reshape(B, Q, 3), (0, 2, 1))
    wt = jnp.concatenate([Wtgt[:3], btgt.reshape(1, 32)], axis=0)
    out_sc = _get_sc_knn()(xyzcat, tpack, cand_flat, wt)
    o = out_sc.reshape(B, Q, 2, 16)
    tgt_keyfeats_cat = jnp.stack([o[:, :, 0], o[:, :, 1]], axis=-1).reshape(B, K_TOP, NCAND, 32)

    src_keypts = src_keypts + 0.0 * jnp.sum(src_keyfeats_cat) + 0.0 * jnp.sum(tgt_keyfeats_cat)
    return src_keypts
